# baseline (speedup 1.0000x reference)
import jax
import jax.numpy as jnp
from jax.experimental import pallas as pl
from jax.experimental.pallas import tpu as pltpu

_TEMP_SCALE = 10.0
_TAU = 0.8
_EPS = 1e-20


def _med3(a, b, c):
    return jnp.maximum(jnp.minimum(a, b), jnp.minimum(jnp.maximum(a, b), c))


def _med5(a, b, c, d, e):
    f = jnp.maximum(jnp.minimum(a, b), jnp.minimum(c, d))
    g = jnp.minimum(jnp.maximum(a, b), jnp.maximum(c, d))
    return _med3(e, f, g)


def _body(x_ref, u_ref, w_ref, b_ref, e_ref, post_ref, mask_ref,
          ind_ref, pad_ref):
    j = pl.program_id(1)
    T = x_ref.shape[1]

    @pl.when(j == 0)
    def _sample():
        h = jnp.dot(w_ref[...], x_ref[...], preferred_element_type=jnp.float32)
        h = h + b_ref[...]
        z = h / _TEMP_SCALE
        m = jnp.max(z, axis=0, keepdims=True)
        p = jnp.exp(z - m)
        p = p / jnp.sum(p, axis=0, keepdims=True)
        post_ref[...] = p
        logits = jnp.log(p)
        g = -jnp.log(-jnp.log(u_ref[...] + _EPS) + _EPS)
        zz = (logits + g) / _TAU
        mm = jnp.max(zz, axis=0, keepdims=True)
        yy = jnp.exp(zz - mm)
        yy = yy / jnp.sum(yy, axis=0, keepdims=True)
        ind_ref[...] = (yy[1:2, :] > yy[0:1, :]).astype(jnp.float32)

    @pl.when(j > 0)
    def _pool():
        masked = ind_ref[...] * e_ref[...]
        x = masked.T
        for _ in range(3):
            pad_ref[2:T + 2, :] = x
            pad_ref[0:1, :] = pad_ref[4:5, :]
            pad_ref[1:2, :] = pad_ref[3:4, :]
            pad_ref[T + 2:T + 3, :] = pad_ref[T:T + 1, :]
            pad_ref[T + 3:T + 4, :] = pad_ref[T - 1:T, :]
            x = _med5(
                pad_ref[0:T, :], pad_ref[1:T + 1, :], pad_ref[2:T + 2, :],
                pad_ref[3:T + 3, :], pad_ref[4:T + 4, :],
            )
        mask_ref[...] = x.T


@jax.jit
def kernel(x, e, u, W, b):
    B, C, T = x.shape
    ut = jnp.transpose(u, (0, 2, 1))
    b2 = jnp.reshape(b, (2, 1))
    CB = 128

    def eblk(i, j):
        return (i, jnp.maximum(j - 1, 0), 0)

    post_t, mask = pl.pallas_call(
        _body,
        grid=(B, 1 + C // CB),
        in_specs=[
            pl.BlockSpec((None, C, T), lambda i, j: (i, 0, 0)),
            pl.BlockSpec((None, 2, T), lambda i, j: (i, 0, 0)),
            pl.BlockSpec((2, C), lambda i, j: (0, 0)),
            pl.BlockSpec((2, 1), lambda i, j: (0, 0)),
            pl.BlockSpec((None, CB, T), eblk),
        ],
        out_specs=[
            pl.BlockSpec((None, 2, T), lambda i, j: (i, 0, 0)),
            pl.BlockSpec((None, CB, T), eblk),
        ],
        out_shape=[
            jax.ShapeDtypeStruct((B, 2, T), jnp.float32),
            jax.ShapeDtypeStruct((B, C, T), jnp.float32),
        ],
        scratch_shapes=[
            pltpu.VMEM((1, T), jnp.float32),
            pltpu.VMEM((T + 8, CB), jnp.float32),
        ],
    )(x, ut, W, b2, e)

    posterior = jnp.transpose(post_t, (0, 2, 1))
    return posterior, mask
